# baseline (speedup 1.0000x reference)
import jax
import jax.numpy as jnp
from jax import lax
from jax.experimental import pallas as pl
from jax.experimental.pallas import tpu as pltpu
from jax.experimental.pallas import tpu_sc as plsc

TOTAL_POS = 64
TOTAL_NEG = 64
NMS_TOPN = 300
NMS_IOU = 0.7

_B = 8
_N = 20000
_NPAD = 20480
_OPAD = 512
_CAP = 512
_CPAD = 528
_NCHUNK = _N // 16


def _sc_extract(lab_hbm, anc_hbm, del_hbm,
                okey_hbm, oay1_hbm, oax1_hbm, oay2_hbm, oax2_hbm,
                ody_hbm, odx_hbm, odh_hbm, odw_hbm,
                key_vm, stage_vm,
                okey_vm, cidx_vm, og_vm,
                histo_vm):
    nc = 2
    w = lax.axis_index("s") * nc + lax.axis_index("c")

    @pl.when(w < _B)
    def _():
        pltpu.sync_copy(lab_hbm.at[w], key_vm)

        lane = lax.broadcasted_iota(jnp.int32, (16,), 0)
        zeros16 = jnp.zeros((16,), jnp.int32)
        ones16 = jnp.ones((16,), jnp.int32)

        def kchunk(i):
            return plsc.bitcast(key_vm[pl.ds(i * 16, 16)], jnp.int32)

        def clr(i, c):
            histo_vm[pl.ds(i * 16, 16)] = zeros16
            return c
        lax.fori_loop(0, 256, clr, 0)

        def h1(i, c):
            b = lax.shift_right_arithmetic(kchunk(i), 23)
            plsc.addupdate_scatter(histo_vm, [b * 16 + lane], ones16)
            return c
        lax.fori_loop(0, _NCHUNK, h1, 0)

        def s1(j, carry):
            e = 255 - j
            found, e_star, acc = carry
            c = jnp.sum(histo_vm[pl.ds(e * 16, 16)])
            hit = jnp.logical_and(jnp.logical_not(found), acc + c > _CAP)
            e_star = jnp.where(hit, e, e_star)
            acc = jnp.where(jnp.logical_or(found, hit), acc, acc + c)
            found = jnp.logical_or(found, hit)
            return found, e_star, acc
        _, e_star, c1_above = lax.fori_loop(
            0, 256, s1, (jnp.bool_(False), jnp.int32(0), jnp.int32(0)))

        lax.fori_loop(0, 256, clr, 0)

        def h2(i, c):
            kv = kchunk(i)
            e = lax.shift_right_arithmetic(kv, 23)
            b = jnp.bitwise_and(lax.shift_right_arithmetic(kv, 15), 255)
            plsc.addupdate_scatter(histo_vm, [b * 16 + lane], ones16,
                                   mask=e == e_star)
            return c
        lax.fori_loop(0, _NCHUNK, h2, 0)

        budget = _CAP - c1_above

        def s2(j, carry):
            b = 255 - j
            found, b_star, acc = carry
            c = jnp.sum(histo_vm[pl.ds(b * 16, 16)])
            hit = jnp.logical_and(jnp.logical_not(found), acc + c > budget)
            b_star = jnp.where(hit, b, b_star)
            acc = jnp.where(jnp.logical_or(found, hit), acc, acc + c)
            found = jnp.logical_or(found, hit)
            return found, b_star, acc
        _, b_star, _ = lax.fori_loop(
            0, 256, s2, (jnp.bool_(False), jnp.int32(-1), jnp.int32(0)))

        t = lax.shift_left(e_star, 23) + lax.shift_left(b_star + 1, 15)

        def ini(i, c):
            okey_vm[pl.ds(i * 16, 16)] = jnp.full((16,), -1, jnp.int32)
            cidx_vm[pl.ds(i * 16, 16)] = zeros16
            return c
        lax.fori_loop(0, _CPAD // 16, ini, 0)

        def comp(i, cnt):
            kv = kchunk(i)
            m = kv >= t
            plsc.store_compressed(okey_vm.at[pl.ds(cnt, 16)], kv, mask=m)
            plsc.store_compressed(cidx_vm.at[pl.ds(cnt, 16)],
                                  i * 16 + lane, mask=m)
            return cnt + jnp.sum(jnp.where(m, 1, 0))
        lax.fori_loop(0, _NCHUNK, comp, jnp.int32(0))

        pltpu.sync_copy(okey_vm, okey_hbm.at[w])

        planes_out = ((oay1_hbm, oax1_hbm, oay2_hbm, oax2_hbm),
                      (ody_hbm, odx_hbm, odh_hbm, odw_hbm))
        for k, blk in enumerate((anc_hbm, del_hbm)):
            pltpu.sync_copy(blk.at[w], stage_vm)
            for c in range(4):
                def g(i, _, c=c):
                    iv = cidx_vm[pl.ds(i * 16, 16)]
                    og_vm[pl.ds(i * 16, 16)] = plsc.load_gather(
                        stage_vm, [iv * 4 + c])
                    return _
                lax.fori_loop(0, _CPAD // 16, g, 0)
                pltpu.sync_copy(og_vm, planes_out[k][c].at[w])


def _extract(lab, anc, dlt):
    mesh = plsc.VectorSubcoreMesh(core_axis_name="c", subcore_axis_name="s")
    out_type = ([jax.ShapeDtypeStruct((_B, _CPAD), jnp.int32)]
                + [jax.ShapeDtypeStruct((_B, _CPAD), jnp.float32)] * 8)
    scratch = ([pltpu.VMEM((_N,), jnp.float32)]
               + [pltpu.VMEM((_N * 4,), jnp.float32)]
               + [pltpu.VMEM((_CPAD,), jnp.int32)] * 2
               + [pltpu.VMEM((_CPAD,), jnp.float32)]
               + [pltpu.VMEM((4096,), jnp.int32)])
    f = pl.kernel(_sc_extract, out_type=out_type, mesh=mesh,
                  scratch_types=scratch,
                  compiler_params=pltpu.CompilerParams(
                      needs_layout_passes=False))
    return f(lab, anc, dlt)


def _scan_body(key0, cay1, cax1, cay2, cax2, cdy, cdx, cdh, cdw,
               oy1, ox1, oy2, ox2, flag):
    aw = cax2[...] - cax1[...]
    ah = cay2[...] - cay1[...]
    acx = cax1[...] + 0.5 * aw
    acy = cay1[...] + 0.5 * ah
    bw = jnp.exp(cdw[...]) * aw
    bh = jnp.exp(cdh[...]) * ah
    bcx = cdx[...] * aw + acx
    bcy = cdy[...] * ah + acy
    vy1 = bcy - 0.5 * bh
    vx1 = bcx - 0.5 * bw
    vy2 = vy1 + bh
    vx2 = vx1 + bw
    areav = (vy2 - vy1) * (vx2 - vx1)

    lane = lax.broadcasted_iota(jnp.int32, (_B, _CAP), 1)
    olane = lax.broadcasted_iota(jnp.int32, (_B, _OPAD), 1)
    NEG = jnp.float32(-3.4e38)
    zf = jnp.zeros((_B, _OPAD), jnp.float32)

    def body(i, carry):
        kv, a1, a2, a3, a4, fl = carry
        m = jnp.max(kv, axis=1, keepdims=True)
        idx = jnp.min(jnp.where(kv == m, lane, jnp.int32(2 ** 30)),
                      axis=1, keepdims=True)
        sel = lane == idx
        sy1 = jnp.max(jnp.where(sel, vy1, NEG), axis=1, keepdims=True)
        sx1 = jnp.max(jnp.where(sel, vx1, NEG), axis=1, keepdims=True)
        sy2 = jnp.max(jnp.where(sel, vy2, NEG), axis=1, keepdims=True)
        sx2 = jnp.max(jnp.where(sel, vx2, NEG), axis=1, keepdims=True)
        a_area = (sy2 - sy1) * (sx2 - sx1)
        iy1 = jnp.maximum(sy1, vy1)
        ix1 = jnp.maximum(sx1, vx1)
        iy2 = jnp.minimum(sy2, vy2)
        ix2 = jnp.minimum(sx2, vx2)
        inter = jnp.maximum(iy2 - iy1, 0.0) * jnp.maximum(ix2 - ix1, 0.0)
        iou = inter / (a_area + areav - inter + 1e-7)
        kv = jnp.where(iou >= NMS_IOU, jnp.int32(-1), kv)
        kv = jnp.where(sel, jnp.int32(-1), kv)
        fl = jnp.maximum(fl, (m == jnp.int32(-1)).astype(jnp.int32))
        hit = olane == i
        a1 = jnp.where(hit, sy1, a1)
        a2 = jnp.where(hit, sx1, a2)
        a3 = jnp.where(hit, sy2, a3)
        a4 = jnp.where(hit, sx2, a4)
        return kv, a1, a2, a3, a4, fl

    kv, a1, a2, a3, a4, fl = lax.fori_loop(
        0, NMS_TOPN, body,
        (key0[...], zf, zf, zf, zf, jnp.zeros((_B, 1), jnp.int32)))
    oy1[...] = a1
    ox1[...] = a2
    oy2[...] = a3
    ox2[...] = a4
    flag[...] = jnp.broadcast_to(fl, (_B, 128))


def _scan(ckey, cay1, cax1, cay2, cax2, cdy, cdx, cdh, cdw):
    return pl.pallas_call(
        _scan_body,
        out_shape=[jax.ShapeDtypeStruct((_B, _OPAD), jnp.float32)] * 4
        + [jax.ShapeDtypeStruct((_B, 128), jnp.int32)],
    )(ckey, cay1, cax1, cay2, cax2, cdy, cdx, cdh, cdw)


def _nms_body(ay1, ax1, ay2, ax2, dy, dx, dh, dw, sc,
              oy1, ox1, oy2, ox2,
              by1, bx1, by2, bx2, area, s):
    aw = ax2[...] - ax1[...]
    ah = ay2[...] - ay1[...]
    acx = ax1[...] + 0.5 * aw
    acy = ay1[...] + 0.5 * ah
    bw = jnp.exp(dw[...]) * aw
    bh = jnp.exp(dh[...]) * ah
    bcx = dx[...] * aw + acx
    bcy = dy[...] * ah + acy
    y1 = bcy - 0.5 * bh
    x1 = bcx - 0.5 * bw
    y2 = y1 + bh
    x2 = x1 + bw
    by1[...] = y1
    bx1[...] = x1
    by2[...] = y2
    bx2[...] = x2
    area[...] = (y2 - y1) * (x2 - x1)
    s[...] = sc[...]

    lane = lax.broadcasted_iota(jnp.int32, (_B, _NPAD), 1)
    olane = lax.broadcasted_iota(jnp.int32, (_B, _OPAD), 1)
    NEG = jnp.float32(-3.4e38)
    SUP = jnp.float32(-1e9)

    def body(i, carry):
        sv = s[...]
        m = jnp.max(sv, axis=1, keepdims=True)
        idx = jnp.min(jnp.where(sv == m, lane, jnp.int32(2 ** 30)),
                      axis=1, keepdims=True)
        sel = lane == idx
        vy1 = by1[...]
        vx1 = bx1[...]
        vy2 = by2[...]
        vx2 = bx2[...]
        sy1 = jnp.max(jnp.where(sel, vy1, NEG), axis=1, keepdims=True)
        sx1 = jnp.max(jnp.where(sel, vx1, NEG), axis=1, keepdims=True)
        sy2 = jnp.max(jnp.where(sel, vy2, NEG), axis=1, keepdims=True)
        sx2 = jnp.max(jnp.where(sel, vx2, NEG), axis=1, keepdims=True)
        a_area = (sy2 - sy1) * (sx2 - sx1)
        iy1 = jnp.maximum(sy1, vy1)
        ix1 = jnp.maximum(sx1, vx1)
        iy2 = jnp.minimum(sy2, vy2)
        ix2 = jnp.minimum(sx2, vx2)
        inter = jnp.maximum(iy2 - iy1, 0.0) * jnp.maximum(ix2 - ix1, 0.0)
        iou = inter / (a_area + area[...] - inter + 1e-7)
        snew = jnp.where(iou >= NMS_IOU, SUP, sv)
        snew = jnp.where(sel, SUP, snew)
        s[...] = snew
        hit = olane == i
        oy1[...] = jnp.where(hit, sy1, oy1[...])
        ox1[...] = jnp.where(hit, sx1, ox1[...])
        oy2[...] = jnp.where(hit, sy2, oy2[...])
        ox2[...] = jnp.where(hit, sx2, ox2[...])
        return carry

    lax.fori_loop(0, NMS_TOPN, body, jnp.int32(0))


def _nms_dense(planes):
    return pl.pallas_call(
        _nms_body,
        out_shape=[jax.ShapeDtypeStruct((_B, _OPAD), jnp.float32)] * 4,
        scratch_shapes=[pltpu.VMEM((_B, _NPAD), jnp.float32)] * 6,
    )(*planes)


def _iou_map(a, b):
    ay1, ax1, ay2, ax2 = jnp.split(a, 4, axis=-1)
    by1, bx1, by2, bx2 = jnp.split(b, 4, axis=-1)
    a_area = (ay2 - ay1) * (ax2 - ax1)
    b_area = (by2 - by1) * (bx2 - bx1)
    iy1 = jnp.maximum(ay1, by1.T)
    ix1 = jnp.maximum(ax1, bx1.T)
    iy2 = jnp.minimum(ay2, by2.T)
    ix2 = jnp.minimum(ax2, bx2.T)
    inter = jnp.maximum(iy2 - iy1, 0.0) * jnp.maximum(ix2 - ix1, 0.0)
    return inter / (a_area + b_area.T - inter + 1e-7)


def _select_single(nms_boxes, gt):
    iou_map = _iou_map(nms_boxes, gt)
    max_gt = jnp.argmax(iou_map, axis=1).astype(jnp.int32)
    merged = jnp.max(iou_map, axis=1)
    order = jnp.argsort(-merged).astype(jnp.int32)
    pos = order[:TOTAL_POS]
    return pos, jnp.take(max_gt, pos, axis=0)


def kernel(rpn_bbox_deltas, rpn_labels, anchors, gt_boxes):
    B, N = anchors.shape[0], anchors.shape[1]
    deltas = rpn_bbox_deltas.reshape(B, N, 4)
    labels = rpn_labels.reshape(B, N)

    compacts = _extract(labels, anchors.reshape(B, N * 4), deltas.reshape(B, N * 4))
    sy1, sx1, sy2, sx2, flag = _scan(*(c[:, :_CAP] for c in compacts))

    def fast_path(_):
        return jnp.stack(
            [sy1[:, :NMS_TOPN], sx1[:, :NMS_TOPN],
             sy2[:, :NMS_TOPN], sx2[:, :NMS_TOPN]], axis=-1)

    def dense_path(_):
        pad = _NPAD - N
        def p0(x):
            return jnp.pad(x, ((0, 0), (0, pad)))
        planes_dense = [
            p0(anchors[..., 0]), p0(anchors[..., 1]),
            p0(anchors[..., 2]), p0(anchors[..., 3]),
            p0(deltas[..., 0]), p0(deltas[..., 1]),
            p0(deltas[..., 2]), p0(deltas[..., 3]),
            jnp.pad(labels, ((0, 0), (0, pad)), constant_values=-3e9),
        ]
        oy1, ox1, oy2, ox2 = _nms_dense(planes_dense)
        return jnp.stack(
            [oy1[:, :NMS_TOPN], ox1[:, :NMS_TOPN],
             oy2[:, :NMS_TOPN], ox2[:, :NMS_TOPN]], axis=-1)

    insufficient = jnp.any(flag != 0)
    nms_bboxes = lax.cond(insufficient, dense_path, fast_path, operand=None)

    bbox_indices, gt_box_indices = jax.vmap(_select_single)(nms_bboxes, gt_boxes)
    pos_roi = jnp.take_along_axis(nms_bboxes, bbox_indices[:, :, None], axis=1)
    neg_roi = jnp.zeros((B, TOTAL_NEG, 4), jnp.float32)
    roi_bboxes = jnp.concatenate([pos_roi, neg_roi], axis=1)
    return (jax.lax.stop_gradient(roi_bboxes), jax.lax.stop_gradient(gt_box_indices))


# speedup vs baseline: 1.3252x; 1.3252x over previous
import jax
import jax.numpy as jnp
from jax import lax
from jax.experimental import pallas as pl
from jax.experimental.pallas import tpu as pltpu
from jax.experimental.pallas import tpu_sc as plsc

TOTAL_POS = 64
TOTAL_NEG = 64
NMS_TOPN = 300
NMS_IOU = 0.7

_B = 8
_N = 20000
_NPAD = 20480
_OPAD = 512
_CAP = 512
_CPAD = 528
_NCHUNK = _N // 16


def _sc_extract(lab_hbm, ay1_hbm, ax1_hbm, ay2_hbm, ax2_hbm,
                dy_hbm, dx_hbm, dh_hbm, dw_hbm,
                okey_hbm, oay1_hbm, oax1_hbm, oay2_hbm, oax2_hbm,
                ody_hbm, odx_hbm, odh_hbm, odw_hbm,
                key_vm, pa_vm, pb_vm,
                okey_vm, cidx_vm, og_vm,
                histo_vm, sem_a, sem_b):
    nc = 2
    w = lax.axis_index("s") * nc + lax.axis_index("c")

    @pl.when(w < _B)
    def _():
        pltpu.sync_copy(lab_hbm.at[w], key_vm)

        lane = lax.broadcasted_iota(jnp.int32, (16,), 0)
        zeros16 = jnp.zeros((16,), jnp.int32)
        ones16 = jnp.ones((16,), jnp.int32)

        def kchunk(i):
            return plsc.bitcast(key_vm[pl.ds(i * 16, 16)], jnp.int32)

        def clr(i, c):
            histo_vm[pl.ds(i * 16, 16)] = zeros16
            return c
        lax.fori_loop(0, 256, clr, 0)

        def h1(i, c):
            b = lax.shift_right_arithmetic(kchunk(i), 23)
            plsc.addupdate_scatter(histo_vm, [b * 16 + lane], ones16)
            return c
        lax.fori_loop(0, _NCHUNK, h1, 0)

        def s1(j, carry):
            e = 255 - j
            found, e_star, acc = carry
            c = jnp.sum(histo_vm[pl.ds(e * 16, 16)])
            hit = jnp.logical_and(jnp.logical_not(found), acc + c > _CAP)
            e_star = jnp.where(hit, e, e_star)
            acc = jnp.where(jnp.logical_or(found, hit), acc, acc + c)
            found = jnp.logical_or(found, hit)
            return found, e_star, acc
        _, e_star, c1_above = lax.fori_loop(
            0, 256, s1, (jnp.bool_(False), jnp.int32(0), jnp.int32(0)))

        lax.fori_loop(0, 256, clr, 0)

        def h2(i, c):
            kv = kchunk(i)
            e = lax.shift_right_arithmetic(kv, 23)
            b = jnp.bitwise_and(lax.shift_right_arithmetic(kv, 15), 255)
            plsc.addupdate_scatter(histo_vm, [b * 16 + lane], ones16,
                                   mask=e == e_star)
            return c
        lax.fori_loop(0, _NCHUNK, h2, 0)

        budget = _CAP - c1_above

        def s2(j, carry):
            b = 255 - j
            found, b_star, acc = carry
            c = jnp.sum(histo_vm[pl.ds(b * 16, 16)])
            hit = jnp.logical_and(jnp.logical_not(found), acc + c > budget)
            b_star = jnp.where(hit, b, b_star)
            acc = jnp.where(jnp.logical_or(found, hit), acc, acc + c)
            found = jnp.logical_or(found, hit)
            return found, b_star, acc
        _, b_star, _ = lax.fori_loop(
            0, 256, s2, (jnp.bool_(False), jnp.int32(-1), jnp.int32(0)))

        t = lax.shift_left(e_star, 23) + lax.shift_left(b_star + 1, 15)

        def ini(i, c):
            okey_vm[pl.ds(i * 16, 16)] = jnp.full((16,), -1, jnp.int32)
            cidx_vm[pl.ds(i * 16, 16)] = zeros16
            return c
        lax.fori_loop(0, _CPAD // 16, ini, 0)

        def comp(i, cnt):
            kv = kchunk(i)
            m = kv >= t
            plsc.store_compressed(okey_vm.at[pl.ds(cnt, 16)], kv, mask=m)
            plsc.store_compressed(cidx_vm.at[pl.ds(cnt, 16)],
                                  i * 16 + lane, mask=m)
            return cnt + jnp.sum(jnp.where(m, 1, 0))
        lax.fori_loop(0, _NCHUNK, comp, jnp.int32(0))

        pltpu.sync_copy(okey_vm, okey_hbm.at[w])

        planes_in = (ay1_hbm, ax1_hbm, ay2_hbm, ax2_hbm,
                     dy_hbm, dx_hbm, dh_hbm, dw_hbm)
        planes_out = (oay1_hbm, oax1_hbm, oay2_hbm, oax2_hbm,
                      ody_hbm, odx_hbm, odh_hbm, odw_hbm)
        bufs = (pa_vm, pb_vm)
        sems = (sem_a, sem_b)
        cps = [pltpu.async_copy(planes_in[0].at[w], bufs[0], sems[0]), None]
        for k in range(8):
            cur = k % 2
            nxt = (k + 1) % 2
            if k < 7:
                cps[nxt] = pltpu.async_copy(
                    planes_in[k + 1].at[w], bufs[nxt], sems[nxt])
            cps[cur].wait()
            buf = bufs[cur]

            def g(i, c):
                iv = cidx_vm[pl.ds(i * 16, 16)]
                og_vm[pl.ds(i * 16, 16)] = plsc.load_gather(buf, [iv])
                return c
            lax.fori_loop(0, _CPAD // 16, g, 0)
            pltpu.sync_copy(og_vm, planes_out[k].at[w])


def _extract(lab, ay1, ax1, ay2, ax2, dy, dx, dh, dw):
    mesh = plsc.VectorSubcoreMesh(core_axis_name="c", subcore_axis_name="s")
    out_type = ([jax.ShapeDtypeStruct((_B, _CPAD), jnp.int32)]
                + [jax.ShapeDtypeStruct((_B, _CPAD), jnp.float32)] * 8)
    scratch = ([pltpu.VMEM((_NPAD,), jnp.float32)] * 3
               + [pltpu.VMEM((_CPAD,), jnp.int32)] * 2
               + [pltpu.VMEM((_CPAD,), jnp.float32)]
               + [pltpu.VMEM((4096,), jnp.int32)]
               + [pltpu.SemaphoreType.DMA] * 2)
    f = pl.kernel(_sc_extract, out_type=out_type, mesh=mesh,
                  scratch_types=scratch,
                  compiler_params=pltpu.CompilerParams(
                      needs_layout_passes=False))
    return f(lab, ay1, ax1, ay2, ax2, dy, dx, dh, dw)


def _scan_body(key0, cay1, cax1, cay2, cax2, cdy, cdx, cdh, cdw,
               oy1, ox1, oy2, ox2, flag):
    aw = cax2[...] - cax1[...]
    ah = cay2[...] - cay1[...]
    acx = cax1[...] + 0.5 * aw
    acy = cay1[...] + 0.5 * ah
    bw = jnp.exp(cdw[...]) * aw
    bh = jnp.exp(cdh[...]) * ah
    bcx = cdx[...] * aw + acx
    bcy = cdy[...] * ah + acy
    vy1 = bcy - 0.5 * bh
    vx1 = bcx - 0.5 * bw
    vy2 = vy1 + bh
    vx2 = vx1 + bw
    areav = (vy2 - vy1) * (vx2 - vx1)

    lane = lax.broadcasted_iota(jnp.int32, (_B, _CAP), 1)
    olane = lax.broadcasted_iota(jnp.int32, (_B, _OPAD), 1)
    NEG = jnp.float32(-3.4e38)
    zf = jnp.zeros((_B, _OPAD), jnp.float32)

    def body(i, carry):
        kv, a1, a2, a3, a4, fl = carry
        m = jnp.max(kv, axis=1, keepdims=True)
        idx = jnp.min(jnp.where(kv == m, lane, jnp.int32(2 ** 30)),
                      axis=1, keepdims=True)
        sel = lane == idx
        sy1 = jnp.max(jnp.where(sel, vy1, NEG), axis=1, keepdims=True)
        sx1 = jnp.max(jnp.where(sel, vx1, NEG), axis=1, keepdims=True)
        sy2 = jnp.max(jnp.where(sel, vy2, NEG), axis=1, keepdims=True)
        sx2 = jnp.max(jnp.where(sel, vx2, NEG), axis=1, keepdims=True)
        a_area = (sy2 - sy1) * (sx2 - sx1)
        iy1 = jnp.maximum(sy1, vy1)
        ix1 = jnp.maximum(sx1, vx1)
        iy2 = jnp.minimum(sy2, vy2)
        ix2 = jnp.minimum(sx2, vx2)
        inter = jnp.maximum(iy2 - iy1, 0.0) * jnp.maximum(ix2 - ix1, 0.0)
        iou = inter / (a_area + areav - inter + 1e-7)
        kv = jnp.where(iou >= NMS_IOU, jnp.int32(-1), kv)
        kv = jnp.where(sel, jnp.int32(-1), kv)
        fl = jnp.maximum(fl, (m == jnp.int32(-1)).astype(jnp.int32))
        hit = olane == i
        a1 = jnp.where(hit, sy1, a1)
        a2 = jnp.where(hit, sx1, a2)
        a3 = jnp.where(hit, sy2, a3)
        a4 = jnp.where(hit, sx2, a4)
        return kv, a1, a2, a3, a4, fl

    kv, a1, a2, a3, a4, fl = lax.fori_loop(
        0, NMS_TOPN, body,
        (key0[...], zf, zf, zf, zf, jnp.zeros((_B, 1), jnp.int32)))
    oy1[...] = a1
    ox1[...] = a2
    oy2[...] = a3
    ox2[...] = a4
    flag[...] = jnp.broadcast_to(fl, (_B, 128))


def _scan(ckey, cay1, cax1, cay2, cax2, cdy, cdx, cdh, cdw):
    return pl.pallas_call(
        _scan_body,
        out_shape=[jax.ShapeDtypeStruct((_B, _OPAD), jnp.float32)] * 4
        + [jax.ShapeDtypeStruct((_B, 128), jnp.int32)],
    )(ckey, cay1, cax1, cay2, cax2, cdy, cdx, cdh, cdw)


def _nms_body(ay1, ax1, ay2, ax2, dy, dx, dh, dw, sc,
              oy1, ox1, oy2, ox2,
              by1, bx1, by2, bx2, area, s):
    aw = ax2[...] - ax1[...]
    ah = ay2[...] - ay1[...]
    acx = ax1[...] + 0.5 * aw
    acy = ay1[...] + 0.5 * ah
    bw = jnp.exp(dw[...]) * aw
    bh = jnp.exp(dh[...]) * ah
    bcx = dx[...] * aw + acx
    bcy = dy[...] * ah + acy
    y1 = bcy - 0.5 * bh
    x1 = bcx - 0.5 * bw
    y2 = y1 + bh
    x2 = x1 + bw
    by1[...] = y1
    bx1[...] = x1
    by2[...] = y2
    bx2[...] = x2
    area[...] = (y2 - y1) * (x2 - x1)
    s[...] = sc[...]

    lane = lax.broadcasted_iota(jnp.int32, (_B, _NPAD), 1)
    olane = lax.broadcasted_iota(jnp.int32, (_B, _OPAD), 1)
    NEG = jnp.float32(-3.4e38)
    SUP = jnp.float32(-1e9)

    def body(i, carry):
        sv = s[...]
        m = jnp.max(sv, axis=1, keepdims=True)
        idx = jnp.min(jnp.where(sv == m, lane, jnp.int32(2 ** 30)),
                      axis=1, keepdims=True)
        sel = lane == idx
        vy1 = by1[...]
        vx1 = bx1[...]
        vy2 = by2[...]
        vx2 = bx2[...]
        sy1 = jnp.max(jnp.where(sel, vy1, NEG), axis=1, keepdims=True)
        sx1 = jnp.max(jnp.where(sel, vx1, NEG), axis=1, keepdims=True)
        sy2 = jnp.max(jnp.where(sel, vy2, NEG), axis=1, keepdims=True)
        sx2 = jnp.max(jnp.where(sel, vx2, NEG), axis=1, keepdims=True)
        a_area = (sy2 - sy1) * (sx2 - sx1)
        iy1 = jnp.maximum(sy1, vy1)
        ix1 = jnp.maximum(sx1, vx1)
        iy2 = jnp.minimum(sy2, vy2)
        ix2 = jnp.minimum(sx2, vx2)
        inter = jnp.maximum(iy2 - iy1, 0.0) * jnp.maximum(ix2 - ix1, 0.0)
        iou = inter / (a_area + area[...] - inter + 1e-7)
        snew = jnp.where(iou >= NMS_IOU, SUP, sv)
        snew = jnp.where(sel, SUP, snew)
        s[...] = snew
        hit = olane == i
        oy1[...] = jnp.where(hit, sy1, oy1[...])
        ox1[...] = jnp.where(hit, sx1, ox1[...])
        oy2[...] = jnp.where(hit, sy2, oy2[...])
        ox2[...] = jnp.where(hit, sx2, ox2[...])
        return carry

    lax.fori_loop(0, NMS_TOPN, body, jnp.int32(0))


def _nms_dense(planes):
    return pl.pallas_call(
        _nms_body,
        out_shape=[jax.ShapeDtypeStruct((_B, _OPAD), jnp.float32)] * 4,
        scratch_shapes=[pltpu.VMEM((_B, _NPAD), jnp.float32)] * 6,
    )(*planes)


def _iou_map(a, b):
    ay1, ax1, ay2, ax2 = jnp.split(a, 4, axis=-1)
    by1, bx1, by2, bx2 = jnp.split(b, 4, axis=-1)
    a_area = (ay2 - ay1) * (ax2 - ax1)
    b_area = (by2 - by1) * (bx2 - bx1)
    iy1 = jnp.maximum(ay1, by1.T)
    ix1 = jnp.maximum(ax1, bx1.T)
    iy2 = jnp.minimum(ay2, by2.T)
    ix2 = jnp.minimum(ax2, bx2.T)
    inter = jnp.maximum(iy2 - iy1, 0.0) * jnp.maximum(ix2 - ix1, 0.0)
    return inter / (a_area + b_area.T - inter + 1e-7)


def _select_single(nms_boxes, gt):
    iou_map = _iou_map(nms_boxes, gt)
    max_gt = jnp.argmax(iou_map, axis=1).astype(jnp.int32)
    merged = jnp.max(iou_map, axis=1)
    order = jnp.argsort(-merged).astype(jnp.int32)
    pos = order[:TOTAL_POS]
    return pos, jnp.take(max_gt, pos, axis=0)


def kernel(rpn_bbox_deltas, rpn_labels, anchors, gt_boxes):
    B, N = anchors.shape[0], anchors.shape[1]
    deltas = rpn_bbox_deltas.reshape(B, N, 4)
    labels = rpn_labels.reshape(B, N)

    pad = _NPAD - N
    def p0(x):
        return jnp.pad(x, ((0, 0), (0, pad)))
    base_planes = [
        p0(anchors[..., 0]), p0(anchors[..., 1]),
        p0(anchors[..., 2]), p0(anchors[..., 3]),
        p0(deltas[..., 0]), p0(deltas[..., 1]),
        p0(deltas[..., 2]), p0(deltas[..., 3]),
    ]
    planes_dense = base_planes + [
        jnp.pad(labels, ((0, 0), (0, pad)), constant_values=-3e9)]

    compacts = _extract(p0(labels), *base_planes)
    sy1, sx1, sy2, sx2, flag = _scan(*(c[:, :_CAP] for c in compacts))

    def fast_path(_):
        return jnp.stack(
            [sy1[:, :NMS_TOPN], sx1[:, :NMS_TOPN],
             sy2[:, :NMS_TOPN], sx2[:, :NMS_TOPN]], axis=-1)

    def dense_path(_):
        oy1, ox1, oy2, ox2 = _nms_dense(planes_dense)
        return jnp.stack(
            [oy1[:, :NMS_TOPN], ox1[:, :NMS_TOPN],
             oy2[:, :NMS_TOPN], ox2[:, :NMS_TOPN]], axis=-1)

    insufficient = jnp.any(flag != 0)
    nms_bboxes = lax.cond(insufficient, dense_path, fast_path, operand=None)

    bbox_indices, gt_box_indices = jax.vmap(_select_single)(nms_bboxes, gt_boxes)
    pos_roi = jnp.take_along_axis(nms_bboxes, bbox_indices[:, :, None], axis=1)
    neg_roi = jnp.zeros((B, TOTAL_NEG, 4), jnp.float32)
    roi_bboxes = jnp.concatenate([pos_roi, neg_roi], axis=1)
    return (jax.lax.stop_gradient(roi_bboxes), jax.lax.stop_gradient(gt_box_indices))


# speedup vs baseline: 1.5209x; 1.1477x over previous
import jax
import jax.numpy as jnp
from jax import lax
from jax.experimental import pallas as pl
from jax.experimental.pallas import tpu as pltpu
from jax.experimental.pallas import tpu_sc as plsc

TOTAL_POS = 64
TOTAL_NEG = 64
NMS_TOPN = 300
NMS_IOU = 0.7

_B = 8
_N = 20000
_NPAD = 20480
_OPAD = 512
_CAP = 512
_CPAD = 528
_NCHUNK = _N // 16


def _sc_extract(lab_hbm, ay1_hbm, ax1_hbm, ay2_hbm, ax2_hbm,
                dy_hbm, dx_hbm, dh_hbm, dw_hbm,
                okey_hbm, oay1_hbm, oax1_hbm, oay2_hbm, oax2_hbm,
                ody_hbm, odx_hbm, odh_hbm, odw_hbm,
                key_vm, pa_vm, pb_vm,
                okey_vm, cidx_vm, og_vm,
                histo_vm, sem_a, sem_b):
    nc = 2
    w = lax.axis_index("s") * nc + lax.axis_index("c")

    @pl.when(w < _B)
    def _():
        pltpu.sync_copy(lab_hbm.at[w], key_vm)

        lane = lax.broadcasted_iota(jnp.int32, (16,), 0)
        zeros16 = jnp.zeros((16,), jnp.int32)
        ones16 = jnp.ones((16,), jnp.int32)

        def kchunk(i):
            return plsc.bitcast(key_vm[pl.ds(i * 16, 16)], jnp.int32)

        def clr(i, c):
            histo_vm[pl.ds(i * 16, 16)] = zeros16
            return c
        lax.fori_loop(0, 256, clr, 0)

        def h1(i, c):
            b = lax.shift_right_arithmetic(kchunk(i), 23)
            plsc.addupdate_scatter(histo_vm, [b * 16 + lane], ones16)
            return c
        lax.fori_loop(0, _NCHUNK, h1, 0)

        def s1(j, carry):
            e = 255 - j
            found, e_star, acc = carry
            c = jnp.sum(histo_vm[pl.ds(e * 16, 16)])
            hit = jnp.logical_and(jnp.logical_not(found), acc + c > _CAP)
            e_star = jnp.where(hit, e, e_star)
            acc = jnp.where(jnp.logical_or(found, hit), acc, acc + c)
            found = jnp.logical_or(found, hit)
            return found, e_star, acc
        _, e_star, c1_above = lax.fori_loop(
            0, 256, s1, (jnp.bool_(False), jnp.int32(0), jnp.int32(0)))

        lax.fori_loop(0, 256, clr, 0)

        def h2(i, c):
            kv = kchunk(i)
            e = lax.shift_right_arithmetic(kv, 23)
            b = jnp.bitwise_and(lax.shift_right_arithmetic(kv, 15), 255)
            plsc.addupdate_scatter(histo_vm, [b * 16 + lane], ones16,
                                   mask=e == e_star)
            return c
        lax.fori_loop(0, _NCHUNK, h2, 0)

        budget = _CAP - c1_above

        def s2(j, carry):
            b = 255 - j
            found, b_star, acc = carry
            c = jnp.sum(histo_vm[pl.ds(b * 16, 16)])
            hit = jnp.logical_and(jnp.logical_not(found), acc + c > budget)
            b_star = jnp.where(hit, b, b_star)
            acc = jnp.where(jnp.logical_or(found, hit), acc, acc + c)
            found = jnp.logical_or(found, hit)
            return found, b_star, acc
        _, b_star, _ = lax.fori_loop(
            0, 256, s2, (jnp.bool_(False), jnp.int32(-1), jnp.int32(0)))

        t = lax.shift_left(e_star, 23) + lax.shift_left(b_star + 1, 15)

        def ini(i, c):
            okey_vm[pl.ds(i * 16, 16)] = jnp.full((16,), -1, jnp.int32)
            cidx_vm[pl.ds(i * 16, 16)] = zeros16
            return c
        lax.fori_loop(0, _CPAD // 16, ini, 0)

        def comp(i, cnt):
            kv = kchunk(i)
            m = kv >= t
            plsc.store_compressed(okey_vm.at[pl.ds(cnt, 16)], kv, mask=m)
            plsc.store_compressed(cidx_vm.at[pl.ds(cnt, 16)],
                                  i * 16 + lane, mask=m)
            return cnt + jnp.sum(jnp.where(m, 1, 0))
        lax.fori_loop(0, _NCHUNK, comp, jnp.int32(0))

        pltpu.sync_copy(okey_vm, okey_hbm.at[w])

        planes_in = (ay1_hbm, ax1_hbm, ay2_hbm, ax2_hbm,
                     dy_hbm, dx_hbm, dh_hbm, dw_hbm)
        planes_out = (oay1_hbm, oax1_hbm, oay2_hbm, oax2_hbm,
                      ody_hbm, odx_hbm, odh_hbm, odw_hbm)
        bufs = (pa_vm, pb_vm)
        sems = (sem_a, sem_b)
        cps = [pltpu.async_copy(planes_in[0].at[w], bufs[0], sems[0]), None]
        for k in range(8):
            cur = k % 2
            nxt = (k + 1) % 2
            if k < 7:
                cps[nxt] = pltpu.async_copy(
                    planes_in[k + 1].at[w], bufs[nxt], sems[nxt])
            cps[cur].wait()
            buf = bufs[cur]

            def g(i, c):
                iv = cidx_vm[pl.ds(i * 16, 16)]
                og_vm[pl.ds(i * 16, 16)] = plsc.load_gather(buf, [iv])
                return c
            lax.fori_loop(0, _CPAD // 16, g, 0)
            pltpu.sync_copy(og_vm, planes_out[k].at[w])


def _extract(lab, ay1, ax1, ay2, ax2, dy, dx, dh, dw):
    mesh = plsc.VectorSubcoreMesh(core_axis_name="c", subcore_axis_name="s")
    out_type = ([jax.ShapeDtypeStruct((_B, _CPAD), jnp.int32)]
                + [jax.ShapeDtypeStruct((_B, _CPAD), jnp.float32)] * 8)
    scratch = ([pltpu.VMEM((_NPAD,), jnp.float32)] * 3
               + [pltpu.VMEM((_CPAD,), jnp.int32)] * 2
               + [pltpu.VMEM((_CPAD,), jnp.float32)]
               + [pltpu.VMEM((4096,), jnp.int32)]
               + [pltpu.SemaphoreType.DMA] * 2)
    f = pl.kernel(_sc_extract, out_type=out_type, mesh=mesh,
                  scratch_types=scratch,
                  compiler_params=pltpu.CompilerParams(
                      needs_layout_passes=False))
    return f(lab, ay1, ax1, ay2, ax2, dy, dx, dh, dw)


def _scan_body(key0, cay1, cax1, cay2, cax2, cdy, cdx, cdh, cdw,
               oy1, ox1, oy2, ox2, flag):
    aw = cax2[...] - cax1[...]
    ah = cay2[...] - cay1[...]
    acx = cax1[...] + 0.5 * aw
    acy = cay1[...] + 0.5 * ah
    bw = jnp.exp(cdw[...]) * aw
    bh = jnp.exp(cdh[...]) * ah
    bcx = cdx[...] * aw + acx
    bcy = cdy[...] * ah + acy
    vy1 = bcy - 0.5 * bh
    vx1 = bcx - 0.5 * bw
    vy2 = vy1 + bh
    vx2 = vx1 + bw
    areav = (vy2 - vy1) * (vx2 - vx1)

    lane = lax.broadcasted_iota(jnp.int32, (_B, _CAP), 1)
    olane = lax.broadcasted_iota(jnp.int32, (_B, _OPAD), 1)
    NEG = jnp.float32(-3.4e38)
    zf = jnp.zeros((_B, _OPAD), jnp.float32)

    def body(i, carry):
        kv, a1, a2, a3, a4, fl = carry
        m = jnp.max(kv, axis=1, keepdims=True)
        sel0 = kv == m
        sy1 = jnp.max(jnp.where(sel0, vy1, NEG), axis=1, keepdims=True)
        sx1 = jnp.max(jnp.where(sel0, vx1, NEG), axis=1, keepdims=True)
        sy2 = jnp.max(jnp.where(sel0, vy2, NEG), axis=1, keepdims=True)
        sx2 = jnp.max(jnp.where(sel0, vx2, NEG), axis=1, keepdims=True)
        tiecnt = jnp.sum(sel0.astype(jnp.int32), axis=1, keepdims=True)

        def repair(_):
            idx = jnp.min(jnp.where(sel0, lane, jnp.int32(2 ** 30)),
                          axis=1, keepdims=True)
            sel = lane == idx
            ry1 = jnp.max(jnp.where(sel, vy1, NEG), axis=1, keepdims=True)
            rx1 = jnp.max(jnp.where(sel, vx1, NEG), axis=1, keepdims=True)
            ry2 = jnp.max(jnp.where(sel, vy2, NEG), axis=1, keepdims=True)
            rx2 = jnp.max(jnp.where(sel, vx2, NEG), axis=1, keepdims=True)
            return ry1, rx1, ry2, rx2

        sy1, sx1, sy2, sx2 = lax.cond(
            jnp.max(tiecnt) > 1, repair,
            lambda _: (sy1, sx1, sy2, sx2), operand=None)
        a_area = (sy2 - sy1) * (sx2 - sx1)
        iy1 = jnp.maximum(sy1, vy1)
        ix1 = jnp.maximum(sx1, vx1)
        iy2 = jnp.minimum(sy2, vy2)
        ix2 = jnp.minimum(sx2, vx2)
        inter = jnp.maximum(iy2 - iy1, 0.0) * jnp.maximum(ix2 - ix1, 0.0)
        iou = inter / (a_area + areav - inter + 1e-7)
        kv = jnp.where(iou >= NMS_IOU, jnp.int32(-1), kv)
        fl = jnp.maximum(fl, (m == jnp.int32(-1)).astype(jnp.int32))
        hit = olane == i
        a1 = jnp.where(hit, sy1, a1)
        a2 = jnp.where(hit, sx1, a2)
        a3 = jnp.where(hit, sy2, a3)
        a4 = jnp.where(hit, sx2, a4)
        return kv, a1, a2, a3, a4, fl

    kv, a1, a2, a3, a4, fl = lax.fori_loop(
        0, NMS_TOPN, body,
        (key0[...], zf, zf, zf, zf, jnp.zeros((_B, 1), jnp.int32)))
    oy1[...] = a1
    ox1[...] = a2
    oy2[...] = a3
    ox2[...] = a4
    flag[...] = jnp.broadcast_to(fl, (_B, 128))


def _scan(ckey, cay1, cax1, cay2, cax2, cdy, cdx, cdh, cdw):
    return pl.pallas_call(
        _scan_body,
        out_shape=[jax.ShapeDtypeStruct((_B, _OPAD), jnp.float32)] * 4
        + [jax.ShapeDtypeStruct((_B, 128), jnp.int32)],
    )(ckey, cay1, cax1, cay2, cax2, cdy, cdx, cdh, cdw)


def _nms_body(ay1, ax1, ay2, ax2, dy, dx, dh, dw, sc,
              oy1, ox1, oy2, ox2,
              by1, bx1, by2, bx2, area, s):
    aw = ax2[...] - ax1[...]
    ah = ay2[...] - ay1[...]
    acx = ax1[...] + 0.5 * aw
    acy = ay1[...] + 0.5 * ah
    bw = jnp.exp(dw[...]) * aw
    bh = jnp.exp(dh[...]) * ah
    bcx = dx[...] * aw + acx
    bcy = dy[...] * ah + acy
    y1 = bcy - 0.5 * bh
    x1 = bcx - 0.5 * bw
    y2 = y1 + bh
    x2 = x1 + bw
    by1[...] = y1
    bx1[...] = x1
    by2[...] = y2
    bx2[...] = x2
    area[...] = (y2 - y1) * (x2 - x1)
    s[...] = sc[...]

    lane = lax.broadcasted_iota(jnp.int32, (_B, _NPAD), 1)
    olane = lax.broadcasted_iota(jnp.int32, (_B, _OPAD), 1)
    NEG = jnp.float32(-3.4e38)
    SUP = jnp.float32(-1e9)

    def body(i, carry):
        sv = s[...]
        m = jnp.max(sv, axis=1, keepdims=True)
        idx = jnp.min(jnp.where(sv == m, lane, jnp.int32(2 ** 30)),
                      axis=1, keepdims=True)
        sel = lane == idx
        vy1 = by1[...]
        vx1 = bx1[...]
        vy2 = by2[...]
        vx2 = bx2[...]
        sy1 = jnp.max(jnp.where(sel, vy1, NEG), axis=1, keepdims=True)
        sx1 = jnp.max(jnp.where(sel, vx1, NEG), axis=1, keepdims=True)
        sy2 = jnp.max(jnp.where(sel, vy2, NEG), axis=1, keepdims=True)
        sx2 = jnp.max(jnp.where(sel, vx2, NEG), axis=1, keepdims=True)
        a_area = (sy2 - sy1) * (sx2 - sx1)
        iy1 = jnp.maximum(sy1, vy1)
        ix1 = jnp.maximum(sx1, vx1)
        iy2 = jnp.minimum(sy2, vy2)
        ix2 = jnp.minimum(sx2, vx2)
        inter = jnp.maximum(iy2 - iy1, 0.0) * jnp.maximum(ix2 - ix1, 0.0)
        iou = inter / (a_area + area[...] - inter + 1e-7)
        snew = jnp.where(iou >= NMS_IOU, SUP, sv)
        snew = jnp.where(sel, SUP, snew)
        s[...] = snew
        hit = olane == i
        oy1[...] = jnp.where(hit, sy1, oy1[...])
        ox1[...] = jnp.where(hit, sx1, ox1[...])
        oy2[...] = jnp.where(hit, sy2, oy2[...])
        ox2[...] = jnp.where(hit, sx2, ox2[...])
        return carry

    lax.fori_loop(0, NMS_TOPN, body, jnp.int32(0))


def _nms_dense(planes):
    return pl.pallas_call(
        _nms_body,
        out_shape=[jax.ShapeDtypeStruct((_B, _OPAD), jnp.float32)] * 4,
        scratch_shapes=[pltpu.VMEM((_B, _NPAD), jnp.float32)] * 6,
    )(*planes)


def _iou_map(a, b):
    ay1, ax1, ay2, ax2 = jnp.split(a, 4, axis=-1)
    by1, bx1, by2, bx2 = jnp.split(b, 4, axis=-1)
    a_area = (ay2 - ay1) * (ax2 - ax1)
    b_area = (by2 - by1) * (bx2 - bx1)
    iy1 = jnp.maximum(ay1, by1.T)
    ix1 = jnp.maximum(ax1, bx1.T)
    iy2 = jnp.minimum(ay2, by2.T)
    ix2 = jnp.minimum(ax2, bx2.T)
    inter = jnp.maximum(iy2 - iy1, 0.0) * jnp.maximum(ix2 - ix1, 0.0)
    return inter / (a_area + b_area.T - inter + 1e-7)


def _select_single(nms_boxes, gt):
    iou_map = _iou_map(nms_boxes, gt)
    max_gt = jnp.argmax(iou_map, axis=1).astype(jnp.int32)
    merged = jnp.max(iou_map, axis=1)
    order = jnp.argsort(-merged).astype(jnp.int32)
    pos = order[:TOTAL_POS]
    return pos, jnp.take(max_gt, pos, axis=0)


def kernel(rpn_bbox_deltas, rpn_labels, anchors, gt_boxes):
    B, N = anchors.shape[0], anchors.shape[1]
    deltas = rpn_bbox_deltas.reshape(B, N, 4)
    labels = rpn_labels.reshape(B, N)

    pad = _NPAD - N
    def p0(x):
        return jnp.pad(x, ((0, 0), (0, pad)))
    base_planes = [
        p0(anchors[..., 0]), p0(anchors[..., 1]),
        p0(anchors[..., 2]), p0(anchors[..., 3]),
        p0(deltas[..., 0]), p0(deltas[..., 1]),
        p0(deltas[..., 2]), p0(deltas[..., 3]),
    ]
    planes_dense = base_planes + [
        jnp.pad(labels, ((0, 0), (0, pad)), constant_values=-3e9)]

    compacts = _extract(p0(labels), *base_planes)
    sy1, sx1, sy2, sx2, flag = _scan(*(c[:, :_CAP] for c in compacts))

    def fast_path(_):
        return jnp.stack(
            [sy1[:, :NMS_TOPN], sx1[:, :NMS_TOPN],
             sy2[:, :NMS_TOPN], sx2[:, :NMS_TOPN]], axis=-1)

    def dense_path(_):
        oy1, ox1, oy2, ox2 = _nms_dense(planes_dense)
        return jnp.stack(
            [oy1[:, :NMS_TOPN], ox1[:, :NMS_TOPN],
             oy2[:, :NMS_TOPN], ox2[:, :NMS_TOPN]], axis=-1)

    insufficient = jnp.any(flag != 0)
    nms_bboxes = lax.cond(insufficient, dense_path, fast_path, operand=None)

    bbox_indices, gt_box_indices = jax.vmap(_select_single)(nms_bboxes, gt_boxes)
    pos_roi = jnp.take_along_axis(nms_bboxes, bbox_indices[:, :, None], axis=1)
    neg_roi = jnp.zeros((B, TOTAL_NEG, 4), jnp.float32)
    roi_bboxes = jnp.concatenate([pos_roi, neg_roi], axis=1)
    return (jax.lax.stop_gradient(roi_bboxes), jax.lax.stop_gradient(gt_box_indices))
